# baseline (speedup 1.0000x reference)
import functools
import math

import numpy as np
import jax
import jax.numpy as jnp
from jax import lax
from jax.experimental import pallas as pl
from jax.experimental.pallas import tpu as pltpu
from jax.experimental.pallas import tpu_sc as plsc

_NL = 8
_BASE_RES = 16
_DESIRED_RES = 192
_EPS = 1e-06
_GROWTH = (_DESIRED_RES / _BASE_RES) ** (1.0 / (_NL - 1))
_SCALES = [int(_BASE_RES * _GROWTH**i) for i in range(_NL)]
_OFFS = [0]
for _r in _SCALES:
    _OFFS.append(_OFFS[-1] + int((_r + 1) ** 3))

_NPTS = 131072
_PROWS = _NPTS // 128
_M = _NL * 8 * _NPTS

_RA = 32


def _coords_body(xyz_ref, idx_ref, w_ref):
    x = (jnp.clip(xyz_ref[0], -1.0, 1.0 - _EPS) + 1.0) / 2.0
    y = (jnp.clip(xyz_ref[1], -1.0, 1.0 - _EPS) + 1.0) / 2.0
    z = (jnp.clip(xyz_ref[2], -1.0, 1.0 - _EPS) + 1.0) / 2.0
    for l in range(_NL):
        s = float(_SCALES[l])
        s1 = _SCALES[l] + 1
        fx, fy, fz = x * s, y * s, z * s
        ix0, iy0, iz0 = (fx.astype(jnp.int32), fy.astype(jnp.int32),
                         fz.astype(jnp.int32))
        ix1, iy1, iz1 = ((fx + 1.0).astype(jnp.int32),
                         (fy + 1.0).astype(jnp.int32),
                         (fz + 1.0).astype(jnp.int32))
        frx = fx - ix0.astype(jnp.float32)
        fry = fy - iy0.astype(jnp.float32)
        frz = fz - iz0.astype(jnp.float32)
        gx0, gy0, gz0 = 1.0 - frx, 1.0 - fry, 1.0 - frz
        for c in range(8):
            bx, by, bz = (c >> 2) & 1, (c >> 1) & 1, c & 1
            ind = ((ix1 if bx else ix0) * (s1 * s1)
                   + (iy1 if by else iy0) * s1
                   + (iz1 if bz else iz0) + _OFFS[l])
            idx_ref[l, c] = ind
            w_ref[l, c] = ((frx if bx else gx0)
                           * (fry if by else gy0)
                           * (frz if bz else gz0))


_coords_call = pl.pallas_call(
    _coords_body,
    grid=(_PROWS // _RA,),
    in_specs=[pl.BlockSpec((3, _RA, 128), lambda j: (0, j, 0))],
    out_specs=[
        pl.BlockSpec((_NL, 8, _RA, 128), lambda j: (0, 0, j, 0)),
        pl.BlockSpec((_NL, 8, _RA, 128), lambda j: (0, 0, j, 0)),
    ],
    out_shape=[
        jax.ShapeDtypeStruct((_NL, 8, _PROWS, 128), jnp.int32),
        jax.ShapeDtypeStruct((_NL, 8, _PROWS, 128), jnp.float32),
    ],
)

_NW = 32
_KS = 32


_NT = 2 * (_PROWS // _KS)


def _gather_body_pipe(tab, idx_hbm, out_hbm, idx_v, rows_v, sems):
    wid = lax.axis_index("s") * 2 + lax.axis_index("c")

    def fire(q, b):
        lc = wid * 2 + q // (_PROWS // _KS)
        t = lax.rem(q, _PROWS // _KS)
        l = lc // 8
        c = lax.rem(lc, 8)
        pltpu.sync_copy(idx_hbm.at[l, c, pl.ds(t * _KS, _KS)], idx_v.at[b])
        for j in range(_KS):
            pltpu.async_copy(tab.at[idx_v.at[b, j]], rows_v.at[b, j],
                             sems.at[b])

    def drain(q, b):
        lc = wid * 2 + q // (_PROWS // _KS)
        t = lax.rem(q, _PROWS // _KS)
        l = lc // 8
        c = lax.rem(lc, 8)
        for j in range(_KS):
            pltpu.make_async_copy(tab.at[idx_v.at[b, j]], rows_v.at[b, j],
                                  sems.at[b]).wait()
        pltpu.sync_copy(rows_v.at[b], out_hbm.at[l, c, pl.ds(t * _KS, _KS)])

    fire(0, 0)

    def body(u, carry):
        q = u * 2
        fire(q + 1, 1)
        drain(q, 0)
        fire(q + 2, 0)
        drain(q + 1, 1)
        return carry

    lax.fori_loop(0, _NT // 2 - 1, body, 0)
    fire(_NT - 1, 1)
    drain(_NT - 2, 0)
    drain(_NT - 1, 1)


@functools.cache
def _gather_call():
    return pl.kernel(
        _gather_body_pipe,
        out_type=jax.ShapeDtypeStruct((_NL, 8, _PROWS, 128), jnp.float32),
        mesh=plsc.VectorSubcoreMesh(
            core_axis_name="c", subcore_axis_name="s", num_cores=2, num_subcores=16
        ),
        scratch_types=[
            pltpu.VMEM((2, _KS, 128), jnp.int32),
            pltpu.VMEM((2, _KS, 128), jnp.float32),
            pltpu.SemaphoreType.DMA((2,)),
        ],
        compiler_params=pltpu.CompilerParams(use_tc_tiling_on_sc=False),
    )

_RB = 128


def _embed_body(params_ref, val_ref, w_ref, out_ref):
    l = pl.program_id(0)
    f = params_ref[l, 0]
    fac = params_ref[l, 1]
    acc_s = jnp.zeros((_RB, 128), jnp.float32)
    acc_c = jnp.zeros((_RB, 128), jnp.float32)
    for c in range(8):
        w = w_ref[0, c]
        v = val_ref[0, c] * f
        acc_s = acc_s + w * jnp.sin(v)
        acc_c = acc_c + w * jnp.cos(v)
    out_ref[0, 0] = acc_s * fac
    out_ref[0, 1] = acc_c * fac


_embed_call = pl.pallas_call(
    _embed_body,
    grid=(_NL, _PROWS // _RB),
    in_specs=[
        pl.BlockSpec(memory_space=pltpu.SMEM),
        pl.BlockSpec((1, 8, _RB, 128), lambda l, j: (l, 0, j, 0)),
        pl.BlockSpec((1, 8, _RB, 128), lambda l, j: (l, 0, j, 0)),
    ],
    out_specs=pl.BlockSpec((1, 2, _RB, 128), lambda l, j: (l, 0, j, 0)),
    out_shape=jax.ShapeDtypeStruct((_NL, 2, _PROWS, 128), jnp.float32),
)


def kernel(xyz, data, alpha_ratio):
    xyzT = xyz.T.reshape(3, _PROWS, 128)
    idx, w = _coords_call(xyzT)
    alpha_scale = jnp.minimum(jnp.asarray(alpha_ratio).astype(jnp.float32), 1.0)
    lvl = jnp.arange(_NL, dtype=jnp.float32)
    factors = (1.0 - jnp.cos(math.pi * jnp.clip(alpha_scale * _NL - lvl, 0.0, 1.0))) * 0.5
    freqs = jnp.asarray((2.0 ** np.linspace(0.0, _NL - 1, _NL)).astype(np.float32))
    params = jnp.stack([freqs, factors], axis=1)
    embs = []
    for ch in range(3):
        val = _gather_call()(data[:, ch], idx)
        embs.append(_embed_call(params, val, w).reshape(_NL, 2, _NPTS))
    emb = jnp.stack(embs, axis=-1)
    val48 = emb.transpose(2, 0, 1, 3).reshape(_NPTS, 48)
    return jnp.concatenate([xyz, val48], axis=1)


# speedup vs baseline: 1.7013x; 1.7013x over previous
import functools
import math

import numpy as np
import jax
import jax.numpy as jnp
from jax import lax
from jax.experimental import pallas as pl
from jax.experimental.pallas import tpu as pltpu
from jax.experimental.pallas import tpu_sc as plsc

_NL = 8
_BASE_RES = 16
_DESIRED_RES = 192
_EPS = 1e-06
_GROWTH = (_DESIRED_RES / _BASE_RES) ** (1.0 / (_NL - 1))
_SCALES = [int(_BASE_RES * _GROWTH**i) for i in range(_NL)]
_OFFS = [0]
for _r in _SCALES:
    _OFFS.append(_OFFS[-1] + int((_r + 1) ** 3))

_NPTS = 131072
_PROWS = _NPTS // 128
_M = _NL * 8 * _NPTS

_RA = 32


def _coords_body(xyz_ref, idx_ref, w_ref):
    x = (jnp.clip(xyz_ref[0], -1.0, 1.0 - _EPS) + 1.0) / 2.0
    y = (jnp.clip(xyz_ref[1], -1.0, 1.0 - _EPS) + 1.0) / 2.0
    z = (jnp.clip(xyz_ref[2], -1.0, 1.0 - _EPS) + 1.0) / 2.0
    for l in range(_NL):
        s = float(_SCALES[l])
        s1 = _SCALES[l] + 1
        fx, fy, fz = x * s, y * s, z * s
        ix0, iy0, iz0 = (fx.astype(jnp.int32), fy.astype(jnp.int32),
                         fz.astype(jnp.int32))
        ix1, iy1, iz1 = ((fx + 1.0).astype(jnp.int32),
                         (fy + 1.0).astype(jnp.int32),
                         (fz + 1.0).astype(jnp.int32))
        frx = fx - ix0.astype(jnp.float32)
        fry = fy - iy0.astype(jnp.float32)
        frz = fz - iz0.astype(jnp.float32)
        gx0, gy0, gz0 = 1.0 - frx, 1.0 - fry, 1.0 - frz
        for c in range(8):
            bx, by, bz = (c >> 2) & 1, (c >> 1) & 1, c & 1
            ind = ((ix1 if bx else ix0) * (s1 * s1)
                   + (iy1 if by else iy0) * s1
                   + (iz1 if bz else iz0) + _OFFS[l])
            idx_ref[l, c] = ind
            w_ref[l, c] = ((frx if bx else gx0)
                           * (fry if by else gy0)
                           * (frz if bz else gz0))


_coords_call = pl.pallas_call(
    _coords_body,
    grid=(_PROWS // _RA,),
    in_specs=[pl.BlockSpec((3, _RA, 128), lambda j: (0, j, 0))],
    out_specs=[
        pl.BlockSpec((_NL, 8, _RA, 128), lambda j: (0, 0, j, 0)),
        pl.BlockSpec((_NL, 8, _RA, 128), lambda j: (0, 0, j, 0)),
    ],
    out_shape=[
        jax.ShapeDtypeStruct((_NL, 8, _PROWS, 128), jnp.int32),
        jax.ShapeDtypeStruct((_NL, 8, _PROWS, 128), jnp.float32),
    ],
)

_NW = 32
_KS = 32


_NLOC = 3
_LSTAGE = (_OFFS[_NLOC] + 7) // 8 * 8
_NCH = _PROWS // _KS
_NT = (_NL - _NLOC) * 8 * _NCH // _NW
_NTL = _NLOC * 8 * _NCH // _NW


def _gather_body_pipe(tab, idx_hbm, out_hbm, idx_v, rows_v, stage_v, sems):
    wid = lax.axis_index("s") * 2 + lax.axis_index("c")
    pltpu.sync_copy(tab.at[pl.ds(0, _LSTAGE)], stage_v)

    def loc(q):
        plane = q // _NCH
        t = lax.rem(q, _NCH)
        return plane // 8, lax.rem(plane, 8), t * _KS

    def fire(k, b):
        l, c, prow = loc(_NLOC * 8 * _NCH + wid * _NT + k)
        pltpu.sync_copy(idx_hbm.at[l, c, pl.ds(prow, _KS)], idx_v.at[b])
        for j in range(_KS):
            pltpu.async_copy(tab.at[idx_v.at[b, j]], rows_v.at[b, j],
                             sems.at[b])

    def drain(k, b):
        l, c, prow = loc(_NLOC * 8 * _NCH + wid * _NT + k)
        for j in range(_KS):
            pltpu.make_async_copy(tab.at[idx_v.at[b, j]], rows_v.at[b, j],
                                  sems.at[b]).wait()
        pltpu.sync_copy(rows_v.at[b], out_hbm.at[l, c, pl.ds(prow, _KS)])

    fire(0, 0)

    def body(u, carry):
        k = u * 2
        fire(k + 1, 1)
        drain(k, 0)
        fire(k + 2, 0)
        drain(k + 1, 1)
        return carry

    lax.fori_loop(0, _NT // 2 - 1, body, 0)
    fire(_NT - 1, 1)
    drain(_NT - 2, 0)
    drain(_NT - 1, 1)

    def lbody(k, carry):
        l, c, prow = loc(wid * _NTL + k)
        pltpu.sync_copy(idx_hbm.at[l, c, pl.ds(prow, _KS)], idx_v.at[0])
        for j in range(_KS):
            for g in range(8):
                iv = idx_v[0, j, pl.ds(g * 16, 16)]
                rows_v[0, j, pl.ds(g * 16, 16)] = plsc.load_gather(
                    stage_v, [iv])
        pltpu.sync_copy(rows_v.at[0], out_hbm.at[l, c, pl.ds(prow, _KS)])
        return carry

    lax.fori_loop(0, _NTL, lbody, 0)


@functools.cache
def _gather_call():
    return pl.kernel(
        _gather_body_pipe,
        out_type=jax.ShapeDtypeStruct((_NL, 8, _PROWS, 128), jnp.float32),
        mesh=plsc.VectorSubcoreMesh(
            core_axis_name="c", subcore_axis_name="s", num_cores=2, num_subcores=16
        ),
        scratch_types=[
            pltpu.VMEM((2, _KS, 128), jnp.int32),
            pltpu.VMEM((2, _KS, 128), jnp.float32),
            pltpu.VMEM((_LSTAGE,), jnp.float32),
            pltpu.SemaphoreType.DMA((2,)),
        ],
        compiler_params=pltpu.CompilerParams(
            use_tc_tiling_on_sc=False, needs_layout_passes=False),
    )

_RB = 128


def _embed_body(params_ref, val_ref, w_ref, out_ref):
    l = pl.program_id(0)
    f = params_ref[l, 0]
    fac = params_ref[l, 1]
    acc_s = jnp.zeros((_RB, 128), jnp.float32)
    acc_c = jnp.zeros((_RB, 128), jnp.float32)
    for c in range(8):
        w = w_ref[0, c]
        v = val_ref[0, c] * f
        acc_s = acc_s + w * jnp.sin(v)
        acc_c = acc_c + w * jnp.cos(v)
    out_ref[0, 0] = acc_s * fac
    out_ref[0, 1] = acc_c * fac


_embed_call = pl.pallas_call(
    _embed_body,
    grid=(_NL, _PROWS // _RB),
    in_specs=[
        pl.BlockSpec(memory_space=pltpu.SMEM),
        pl.BlockSpec((1, 8, _RB, 128), lambda l, j: (l, 0, j, 0)),
        pl.BlockSpec((1, 8, _RB, 128), lambda l, j: (l, 0, j, 0)),
    ],
    out_specs=pl.BlockSpec((1, 2, _RB, 128), lambda l, j: (l, 0, j, 0)),
    out_shape=jax.ShapeDtypeStruct((_NL, 2, _PROWS, 128), jnp.float32),
)


def kernel(xyz, data, alpha_ratio):
    xyzT = xyz.T.reshape(3, _PROWS, 128)
    idx, w = _coords_call(xyzT)
    alpha_scale = jnp.minimum(jnp.asarray(alpha_ratio).astype(jnp.float32), 1.0)
    lvl = jnp.arange(_NL, dtype=jnp.float32)
    factors = (1.0 - jnp.cos(math.pi * jnp.clip(alpha_scale * _NL - lvl, 0.0, 1.0))) * 0.5
    freqs = jnp.asarray((2.0 ** np.linspace(0.0, _NL - 1, _NL)).astype(np.float32))
    params = jnp.stack([freqs, factors], axis=1)
    embs = []
    for ch in range(3):
        val = _gather_call()(data[:, ch], idx)
        embs.append(_embed_call(params, val, w).reshape(_NL, 2, _NPTS))
    emb = jnp.stack(embs, axis=-1)
    val48 = emb.transpose(2, 0, 1, 3).reshape(_NPTS, 48)
    return jnp.concatenate([xyz, val48], axis=1)


# speedup vs baseline: 1.7014x; 1.0000x over previous
import functools
import math

import numpy as np
import jax
import jax.numpy as jnp
from jax import lax
from jax.experimental import pallas as pl
from jax.experimental.pallas import tpu as pltpu
from jax.experimental.pallas import tpu_sc as plsc

_NL = 8
_BASE_RES = 16
_DESIRED_RES = 192
_EPS = 1e-06
_GROWTH = (_DESIRED_RES / _BASE_RES) ** (1.0 / (_NL - 1))
_SCALES = [int(_BASE_RES * _GROWTH**i) for i in range(_NL)]
_OFFS = [0]
for _r in _SCALES:
    _OFFS.append(_OFFS[-1] + int((_r + 1) ** 3))

_NPTS = 131072
_PROWS = _NPTS // 128
_M = _NL * 8 * _NPTS

_RA = 32


def _coords_body(xyz_ref, idx_ref, w_ref):
    x = (jnp.clip(xyz_ref[0], -1.0, 1.0 - _EPS) + 1.0) / 2.0
    y = (jnp.clip(xyz_ref[1], -1.0, 1.0 - _EPS) + 1.0) / 2.0
    z = (jnp.clip(xyz_ref[2], -1.0, 1.0 - _EPS) + 1.0) / 2.0
    for l in range(_NL):
        s = float(_SCALES[l])
        s1 = _SCALES[l] + 1
        fx, fy, fz = x * s, y * s, z * s
        ix0, iy0, iz0 = (fx.astype(jnp.int32), fy.astype(jnp.int32),
                         fz.astype(jnp.int32))
        ix1, iy1, iz1 = ((fx + 1.0).astype(jnp.int32),
                         (fy + 1.0).astype(jnp.int32),
                         (fz + 1.0).astype(jnp.int32))
        frx = fx - ix0.astype(jnp.float32)
        fry = fy - iy0.astype(jnp.float32)
        frz = fz - iz0.astype(jnp.float32)
        gx0, gy0, gz0 = 1.0 - frx, 1.0 - fry, 1.0 - frz
        for c in range(8):
            bx, by, bz = (c >> 2) & 1, (c >> 1) & 1, c & 1
            ind = ((ix1 if bx else ix0) * (s1 * s1)
                   + (iy1 if by else iy0) * s1
                   + (iz1 if bz else iz0) + _OFFS[l])
            idx_ref[l, c] = ind
            w_ref[l, c] = ((frx if bx else gx0)
                           * (fry if by else gy0)
                           * (frz if bz else gz0))


_coords_call = pl.pallas_call(
    _coords_body,
    grid=(_PROWS // _RA,),
    in_specs=[pl.BlockSpec((3, _RA, 128), lambda j: (0, j, 0))],
    out_specs=[
        pl.BlockSpec((_NL, 8, _RA, 128), lambda j: (0, 0, j, 0)),
        pl.BlockSpec((_NL, 8, _RA, 128), lambda j: (0, 0, j, 0)),
    ],
    out_shape=[
        jax.ShapeDtypeStruct((_NL, 8, _PROWS, 128), jnp.int32),
        jax.ShapeDtypeStruct((_NL, 8, _PROWS, 128), jnp.float32),
    ],
)

_NW = 32
_KS = 32


_NLOC = 3
_LSTAGE = (_OFFS[_NLOC] + 7) // 8 * 8
_NCH = _PROWS // _KS
_NT = (_NL - _NLOC) * 8 * _NCH // _NW
_NTL = _NLOC * 8 * _NCH // _NW


def _gather_body_pipe(tab, idx_hbm, out_hbm, idx_v, rows_v, stage_v, sems):
    wid = lax.axis_index("s") * 2 + lax.axis_index("c")
    pltpu.sync_copy(tab.at[pl.ds(0, _LSTAGE)], stage_v)

    def loc(q):
        plane = q // _NCH
        t = lax.rem(q, _NCH)
        return plane // 8, lax.rem(plane, 8), t * _KS

    def fire(k, b):
        l, c, prow = loc(_NLOC * 8 * _NCH + wid * _NT + k)
        pltpu.sync_copy(idx_hbm.at[l, c, pl.ds(prow, _KS)], idx_v.at[b])
        for j in range(_KS):
            pltpu.async_copy(tab.at[idx_v.at[b, j]], rows_v.at[b, j],
                             sems.at[b])

    def drain(k, b):
        l, c, prow = loc(_NLOC * 8 * _NCH + wid * _NT + k)
        for j in range(_KS):
            pltpu.make_async_copy(tab.at[idx_v.at[b, j]], rows_v.at[b, j],
                                  sems.at[b]).wait()
        pltpu.sync_copy(rows_v.at[b], out_hbm.at[l, c, pl.ds(prow, _KS)])

    fire(0, 0)

    def body(u, carry):
        k = u * 2
        fire(k + 1, 1)
        drain(k, 0)
        fire(k + 2, 0)
        drain(k + 1, 1)
        return carry

    lax.fori_loop(0, _NT // 2 - 1, body, 0)
    fire(_NT - 1, 1)
    drain(_NT - 2, 0)
    drain(_NT - 1, 1)

    def lbody(k, carry):
        l, c, prow = loc(wid * _NTL + k)
        pltpu.sync_copy(idx_hbm.at[l, c, pl.ds(prow, _KS)], idx_v.at[0])
        for j in range(_KS):
            for g in range(8):
                iv = idx_v[0, j, pl.ds(g * 16, 16)]
                rows_v[0, j, pl.ds(g * 16, 16)] = plsc.load_gather(
                    stage_v, [iv])
        pltpu.sync_copy(rows_v.at[0], out_hbm.at[l, c, pl.ds(prow, _KS)])
        return carry

    lax.fori_loop(0, _NTL, lbody, 0)


@functools.cache
def _gather_call():
    return pl.kernel(
        _gather_body_pipe,
        out_type=jax.ShapeDtypeStruct((_NL, 8, _PROWS, 128), jnp.float32),
        mesh=plsc.VectorSubcoreMesh(
            core_axis_name="c", subcore_axis_name="s", num_cores=2, num_subcores=16
        ),
        scratch_types=[
            pltpu.VMEM((2, _KS, 128), jnp.int32),
            pltpu.VMEM((2, _KS, 128), jnp.float32),
            pltpu.VMEM((_LSTAGE,), jnp.float32),
            pltpu.SemaphoreType.DMA((2,)),
        ],
        compiler_params=pltpu.CompilerParams(
            use_tc_tiling_on_sc=False, needs_layout_passes=False),
    )

_RB = 128


def _embed_body(params_ref, val_ref, w_ref, out_ref):
    l = pl.program_id(0)
    f = params_ref[l, 0]
    fac = params_ref[l, 1]
    acc_s = jnp.zeros((_RB, 128), jnp.float32)
    acc_c = jnp.zeros((_RB, 128), jnp.float32)
    for c in range(8):
        w = w_ref[0, c]
        v = val_ref[0, c] * f
        acc_s = acc_s + w * jnp.sin(v)
        acc_c = acc_c + w * jnp.cos(v)
    out_ref[0, 0] = acc_s * fac
    out_ref[0, 1] = acc_c * fac


_embed_call = pl.pallas_call(
    _embed_body,
    grid=(_NL, _PROWS // _RB),
    in_specs=[
        pl.BlockSpec(memory_space=pltpu.SMEM),
        pl.BlockSpec((1, 8, _RB, 128), lambda l, j: (l, 0, j, 0)),
        pl.BlockSpec((1, 8, _RB, 128), lambda l, j: (l, 0, j, 0)),
    ],
    out_specs=pl.BlockSpec((1, 2, _RB, 128), lambda l, j: (l, 0, j, 0)),
    out_shape=jax.ShapeDtypeStruct((_NL, 2, _PROWS, 128), jnp.float32),
)


def kernel(xyz, data, alpha_ratio):
    xyzT = xyz.T.reshape(3, _PROWS, 128)
    idx, w = _coords_call(xyzT)
    alpha_scale = jnp.minimum(jnp.asarray(alpha_ratio).astype(jnp.float32), 1.0)
    lvl = jnp.arange(_NL, dtype=jnp.float32)
    factors = (1.0 - jnp.cos(math.pi * jnp.clip(alpha_scale * _NL - lvl, 0.0, 1.0))) * 0.5
    freqs = jnp.asarray((2.0 ** np.linspace(0.0, _NL - 1, _NL)).astype(np.float32))
    params = jnp.stack([freqs, factors], axis=1)
    dataT = data.T
    embs = []
    for ch in range(3):
        val = _gather_call()(dataT[ch], idx)
        embs.append(_embed_call(params, val, w).reshape(_NL, 2, _NPTS))
    emb = jnp.stack(embs, axis=-1)
    val48 = emb.transpose(2, 0, 1, 3).reshape(_NPTS, 48)
    return jnp.concatenate([xyz, val48], axis=1)


# speedup vs baseline: 1.7883x; 1.0510x over previous
import functools
import math

import numpy as np
import jax
import jax.numpy as jnp
from jax import lax
from jax.experimental import pallas as pl
from jax.experimental.pallas import tpu as pltpu
from jax.experimental.pallas import tpu_sc as plsc

_NL = 8
_BASE_RES = 16
_DESIRED_RES = 192
_EPS = 1e-06
_GROWTH = (_DESIRED_RES / _BASE_RES) ** (1.0 / (_NL - 1))
_SCALES = [int(_BASE_RES * _GROWTH**i) for i in range(_NL)]
_OFFS = [0]
for _r in _SCALES:
    _OFFS.append(_OFFS[-1] + int((_r + 1) ** 3))

_NPTS = 131072
_PROWS = _NPTS // 128
_M = _NL * 8 * _NPTS

_RA = 32


def _coords_body(xyz_ref, idx_ref, w_ref):
    x = (jnp.clip(xyz_ref[0], -1.0, 1.0 - _EPS) + 1.0) / 2.0
    y = (jnp.clip(xyz_ref[1], -1.0, 1.0 - _EPS) + 1.0) / 2.0
    z = (jnp.clip(xyz_ref[2], -1.0, 1.0 - _EPS) + 1.0) / 2.0
    for l in range(_NL):
        s = float(_SCALES[l])
        s1 = _SCALES[l] + 1
        fx, fy, fz = x * s, y * s, z * s
        ix0, iy0, iz0 = (fx.astype(jnp.int32), fy.astype(jnp.int32),
                         fz.astype(jnp.int32))
        ix1, iy1, iz1 = ((fx + 1.0).astype(jnp.int32),
                         (fy + 1.0).astype(jnp.int32),
                         (fz + 1.0).astype(jnp.int32))
        frx = fx - ix0.astype(jnp.float32)
        fry = fy - iy0.astype(jnp.float32)
        frz = fz - iz0.astype(jnp.float32)
        gx0, gy0, gz0 = 1.0 - frx, 1.0 - fry, 1.0 - frz
        for c in range(8):
            bx, by, bz = (c >> 2) & 1, (c >> 1) & 1, c & 1
            ind = ((ix1 if bx else ix0) * (s1 * s1)
                   + (iy1 if by else iy0) * s1
                   + (iz1 if bz else iz0) + _OFFS[l])
            idx_ref[l, c] = ind
            w_ref[l, c] = ((frx if bx else gx0)
                           * (fry if by else gy0)
                           * (frz if bz else gz0))


_coords_call = pl.pallas_call(
    _coords_body,
    grid=(_PROWS // _RA,),
    in_specs=[pl.BlockSpec((3, _RA, 128), lambda j: (0, j, 0))],
    out_specs=[
        pl.BlockSpec((_NL, 8, _RA, 128), lambda j: (0, 0, j, 0)),
        pl.BlockSpec((_NL, 8, _RA, 128), lambda j: (0, 0, j, 0)),
    ],
    out_shape=[
        jax.ShapeDtypeStruct((_NL, 8, _PROWS, 128), jnp.int32),
        jax.ShapeDtypeStruct((_NL, 8, _PROWS, 128), jnp.float32),
    ],
)

_NW = 32
_KS = 32


_NLOC = 3
_LSTAGE = (_OFFS[_NLOC] + 7) // 8 * 8
_NCH = _PROWS // _KS
_NT = (_NL - _NLOC) * 8 * _NCH // _NW
_NTL = _NLOC * 8 * _NCH // _NW


def _gather_body_pipe(tab, idx_hbm, out_hbm, idx_v, rows_v, stage_v, sems):
    wid = lax.axis_index("s") * 2 + lax.axis_index("c")
    pltpu.sync_copy(tab.at[pl.ds(0, _LSTAGE)], stage_v)

    def loc(q):
        plane = q // _NCH
        t = lax.rem(q, _NCH)
        return plane // 8, lax.rem(plane, 8), t * _KS

    def fire(k, b):
        l, c, prow = loc(_NLOC * 8 * _NCH + wid * _NT + k)
        pltpu.sync_copy(idx_hbm.at[l, c, pl.ds(prow, _KS)], idx_v.at[b])
        for j in range(_KS):
            pltpu.async_copy(tab.at[idx_v.at[b, j]], rows_v.at[b, j],
                             sems.at[b])

    def drain(k, b):
        l, c, prow = loc(_NLOC * 8 * _NCH + wid * _NT + k)
        for j in range(_KS):
            pltpu.make_async_copy(tab.at[idx_v.at[b, j]], rows_v.at[b, j],
                                  sems.at[b]).wait()
        pltpu.sync_copy(rows_v.at[b], out_hbm.at[l, c, pl.ds(prow, _KS)])

    fire(0, 0)

    def body(u, carry):
        k = u * 2
        fire(k + 1, 1)
        drain(k, 0)
        fire(k + 2, 0)
        drain(k + 1, 1)
        return carry

    lax.fori_loop(0, _NT // 2 - 1, body, 0)
    fire(_NT - 1, 1)
    drain(_NT - 2, 0)
    drain(_NT - 1, 1)

    def lbody(k, carry):
        l, c, prow = loc(wid * _NTL + k)
        pltpu.sync_copy(idx_hbm.at[l, c, pl.ds(prow, _KS)], idx_v.at[0])
        for j in range(_KS):
            for g in range(8):
                iv = idx_v[0, j, pl.ds(g * 16, 16)]
                rows_v[0, j, pl.ds(g * 16, 16)] = plsc.load_gather(
                    stage_v, [iv])
        pltpu.sync_copy(rows_v.at[0], out_hbm.at[l, c, pl.ds(prow, _KS)])
        return carry

    lax.fori_loop(0, _NTL, lbody, 0)


@functools.cache
def _gather_call():
    return pl.kernel(
        _gather_body_pipe,
        out_type=jax.ShapeDtypeStruct((_NL, 8, _PROWS, 128), jnp.float32),
        mesh=plsc.VectorSubcoreMesh(
            core_axis_name="c", subcore_axis_name="s", num_cores=2, num_subcores=16
        ),
        scratch_types=[
            pltpu.VMEM((2, _KS, 128), jnp.int32),
            pltpu.VMEM((2, _KS, 128), jnp.float32),
            pltpu.VMEM((_LSTAGE,), jnp.float32),
            pltpu.SemaphoreType.DMA((2,)),
        ],
        compiler_params=pltpu.CompilerParams(
            use_tc_tiling_on_sc=False, needs_layout_passes=False),
    )

_RB = 128


def _embed_body(params_ref, val_ref, w_ref, out_ref):
    l = pl.program_id(0)
    f = params_ref[l, 0]
    fac = params_ref[l, 1]
    acc_s = jnp.zeros((_RB, 128), jnp.float32)
    acc_c = jnp.zeros((_RB, 128), jnp.float32)
    for c in range(8):
        w = w_ref[0, c]
        v = val_ref[0, c] * f
        acc_s = acc_s + w * jnp.sin(v)
        acc_c = acc_c + w * jnp.cos(v)
    out_ref[0, 0] = acc_s * fac
    out_ref[0, 1] = acc_c * fac


_embed_call = pl.pallas_call(
    _embed_body,
    grid=(_NL, _PROWS // _RB),
    in_specs=[
        pl.BlockSpec(memory_space=pltpu.SMEM),
        pl.BlockSpec((1, 8, _RB, 128), lambda l, j: (l, 0, j, 0)),
        pl.BlockSpec((1, 8, _RB, 128), lambda l, j: (l, 0, j, 0)),
    ],
    out_specs=pl.BlockSpec((1, 2, _RB, 128), lambda l, j: (l, 0, j, 0)),
    out_shape=jax.ShapeDtypeStruct((_NL, 2, _PROWS, 128), jnp.float32),
)


def kernel(xyz, data, alpha_ratio):
    xyzT = xyz.T.reshape(3, _PROWS, 128)
    idx, w = _coords_call(xyzT)
    alpha_scale = jnp.minimum(jnp.asarray(alpha_ratio).astype(jnp.float32), 1.0)
    lvl = jnp.arange(_NL, dtype=jnp.float32)
    factors = (1.0 - jnp.cos(math.pi * jnp.clip(alpha_scale * _NL - lvl, 0.0, 1.0))) * 0.5
    freqs = jnp.asarray((2.0 ** np.linspace(0.0, _NL - 1, _NL)).astype(np.float32))
    params = jnp.stack([freqs, factors], axis=1)
    v = data.shape[0]
    vpad = (v + 127) // 128 * 128
    planar = jnp.pad(data, ((0, vpad - v), (0, 0))).T.reshape(3, vpad // 128, 128)
    embs = []
    for ch in range(3):
        val = _gather_call()(planar[ch].reshape(vpad), idx)
        embs.append(_embed_call(params, val, w).reshape(_NL, 2, _NPTS))
    emb = jnp.stack(embs, axis=-1)
    val48 = emb.transpose(2, 0, 1, 3).reshape(_NPTS, 48)
    return jnp.concatenate([xyz, val48], axis=1)
